# baseline (device time: 79073 ns/iter reference)
import jax
import jax.numpy as jnp
from jax import lax
from jax.experimental import pallas as pl
from jax.experimental.pallas import tpu as pltpu

NZ = 4
P = 560
N_COL = 1024
N_ROWS = 2048


def _a2av(x, dest_col):
    n, ncol = x.shape

    def body(x_ref, dest_ref, out_ref,
             send_ref, recv_ref, xbf_ref, cnt_send_ref, cnt_recv_ref,
             data_send_sems, data_recv_sems, cnt_send_sems, cnt_recv_sems):
        my_x = lax.axis_index("x")
        my_y = lax.axis_index("y")
        my_z = lax.axis_index("z")

        barrier_sem = pltpu.get_barrier_semaphore()
        for d in range(1, NZ):
            pl.semaphore_signal(
                barrier_sem, inc=1,
                device_id=(my_x, my_y, (my_z + d) % NZ),
                device_id_type=pl.DeviceIdType.MESH,
            )
        pl.semaphore_wait(barrier_sem, NZ - 1)

        dest_c = dest_ref[...]
        mask = dest_c == lax.broadcasted_iota(jnp.int32, (n, 128), 1)
        counts_row = jnp.sum(mask.astype(jnp.float32), axis=0, keepdims=True
                             ).astype(jnp.int32)
        cnt_send_ref[...] = jnp.where(
            lax.broadcasted_iota(jnp.int32, (8, 128), 0) == 0, counts_row, 0)
        cnt_rdmas = []
        for d in range(1, NZ):
            peer = (my_z + d) % NZ
            cnt_rdma = pltpu.make_async_remote_copy(
                src_ref=cnt_send_ref,
                dst_ref=cnt_recv_ref.at[d - 1],
                send_sem=cnt_send_sems.at[d - 1],
                recv_sem=cnt_recv_sems.at[d - 1],
                device_id=(my_x, my_y, peer),
                device_id_type=pl.DeviceIdType.MESH,
            )
            cnt_rdma.start()
            cnt_rdmas.append(cnt_rdma)

        NB, B = 8, n // 8
        tri_b = (lax.broadcasted_iota(jnp.int32, (B, B), 1)
                 < lax.broadcasted_iota(jnp.int32, (B, B), 0)
                 ).astype(jnp.bfloat16)
        base = jnp.zeros((1, 128), jnp.float32)
        rank_blocks = []
        for blk in range(NB):
            mask_blk = mask[blk * B:(blk + 1) * B, :]
            cum_blk = jnp.dot(tri_b, mask_blk.astype(jnp.bfloat16),
                              preferred_element_type=jnp.float32) + base
            rank_blocks.append(
                jnp.sum(jnp.where(mask_blk, cum_blk, 0.0),
                        axis=1, keepdims=True))
            base = base + jnp.sum(mask_blk.astype(jnp.float32),
                                  axis=0, keepdims=True)
        rank_c = jnp.concatenate(rank_blocks, axis=0).astype(jnp.int32)
        key_c = dest_c * P + rank_c
        iota_blk = lax.broadcasted_iota(jnp.int32, (n, P), 1)
        xbf_ref[...] = x_ref[...].astype(jnp.bfloat16)

        def pack_chunk(t):
            sel = (key_c == t * P + iota_blk).astype(jnp.bfloat16)
            chunk = lax.dot_general(
                sel, xbf_ref[...], dimension_numbers=(((0,), (0,)), ((), ())),
                preferred_element_type=jnp.float32)
            return chunk.astype(jnp.bfloat16)

        data_rdmas = {}
        for d in range(NZ - 1, 0, -1):
            peer = (my_z + d) % NZ
            send_ref[d - 1] = pack_chunk(peer)
            data_rdma = pltpu.make_async_remote_copy(
                src_ref=send_ref.at[d - 1],
                dst_ref=recv_ref.at[d],
                send_sem=data_send_sems.at[d - 1],
                recv_sem=data_recv_sems.at[d - 1],
                device_id=(my_x, my_y, peer),
                device_id_type=pl.DeviceIdType.MESH,
            )
            data_rdma.start()
            data_rdmas[d] = data_rdma

        recv_ref[0] = pack_chunk(my_z)

        for cnt_rdma in cnt_rdmas:
            cnt_rdma.wait()
        col_mask = lax.broadcasted_iota(jnp.int32, (8, 128), 1) == my_z
        row_mask = lax.broadcasted_iota(jnp.int32, (8, 128), 0) == 0
        lmask = col_mask & row_mask

        def _len_of(plane):
            return jnp.sum(jnp.where(lmask, plane, 0))

        l_by_d = [_len_of(cnt_send_ref[...])] + [
            _len_of(cnt_recv_ref[d - 1]) for d in range(1, NZ)
        ]

        len_src, slot_src = [], []
        for s in range(NZ):
            d_s = (my_z - s) % NZ
            ln = l_by_d[0]
            for d in range(1, NZ):
                ln = jnp.where(d_s == d, l_by_d[d], ln)
            len_src.append(ln)
            slot_src.append(d_s)
        starts = [jnp.int32(0)]
        for s in range(1, NZ):
            starts.append(starts[s - 1] + len_src[s - 1])

        j2 = lax.broadcasted_iota(jnp.int32, (n, 1), 0)
        s_idx = jnp.zeros((n, 1), jnp.int32)
        for s in range(1, NZ):
            s_idx = s_idx + (j2 >= starts[s]).astype(jnp.int32)
        start_j = jnp.full((n, 1), starts[0], jnp.int32)
        slot_j = jnp.full((n, 1), slot_src[0], jnp.int32)
        for s in range(1, NZ):
            sel_s = s_idx == s
            start_j = jnp.where(sel_s, starts[s], start_j)
            slot_j = jnp.where(sel_s, slot_src[s], slot_j)
        col_j = slot_j * P + (j2 - start_j)

        def partial(m):
            gsel = (col_j == m * P + iota_blk).astype(jnp.bfloat16)
            return jnp.dot(gsel, recv_ref[m],
                           preferred_element_type=jnp.float32
                           ).astype(jnp.bfloat16)

        out_ref[...] = partial(0)
        for d in range(1, NZ):
            data_rdmas[d].wait()
            out_ref[...] = out_ref[...] + partial(d)

    return pl.pallas_call(
        body,
        out_shape=jax.ShapeDtypeStruct((n, ncol), jnp.bfloat16),
        in_specs=[
            pl.BlockSpec(memory_space=pltpu.VMEM),
            pl.BlockSpec(memory_space=pltpu.VMEM),
        ],
        out_specs=pl.BlockSpec(memory_space=pltpu.VMEM),
        scratch_shapes=[
            pltpu.VMEM((NZ - 1, P, N_COL), jnp.bfloat16),
            pltpu.VMEM((NZ, P, N_COL), jnp.bfloat16),
            pltpu.VMEM((N_ROWS, N_COL), jnp.bfloat16),
            pltpu.VMEM((8, 128), jnp.int32),
            pltpu.VMEM((NZ - 1, 8, 128), jnp.int32),
            pltpu.SemaphoreType.DMA((NZ - 1,)),
            pltpu.SemaphoreType.DMA((NZ - 1,)),
            pltpu.SemaphoreType.DMA((NZ - 1,)),
            pltpu.SemaphoreType.DMA((NZ - 1,)),
        ],
        compiler_params=pltpu.CompilerParams(
            collective_id=0, vmem_limit_bytes=56 * 1024 * 1024),
    )(x, dest_col)


def kernel(x, dest):
    n, _ = x.shape
    return _a2av(x, dest.astype(jnp.int32).reshape(n, 1))


# device time: 32955 ns/iter; 2.3994x vs baseline; 2.3994x over previous
import os

import jax
import jax.numpy as jnp
from jax import lax
from jax.experimental import pallas as pl
from jax.experimental.pallas import tpu as pltpu

COMM = os.environ.get("NO_COMM") != "1"

NZ = 4
P = 560
N_COL = 1024
N_ROWS = 2048


def _a2av(x, dest_col):
    n, ncol = x.shape

    def body(x_ref, dest_ref, out_ref,
             send_ref, recv_ref, xbf_ref, cnt_send_ref, cnt_recv_ref,
             data_send_sems, data_recv_sems, cnt_send_sems, cnt_recv_sems):
        my_x = lax.axis_index("x")
        my_y = lax.axis_index("y")
        my_z = lax.axis_index("z")

        if COMM:
            barrier_sem = pltpu.get_barrier_semaphore()
            for d in range(1, NZ):
                pl.semaphore_signal(
                    barrier_sem, inc=1,
                    device_id=(my_x, my_y, (my_z + d) % NZ),
                    device_id_type=pl.DeviceIdType.MESH,
                )
            pl.semaphore_wait(barrier_sem, NZ - 1)

        dest_c = dest_ref[...]
        mask = dest_c == lax.broadcasted_iota(jnp.int32, (n, 128), 1)
        counts_row = jnp.sum(mask.astype(jnp.float32), axis=0, keepdims=True
                             ).astype(jnp.int32)
        cnt_send_ref[...] = jnp.where(
            lax.broadcasted_iota(jnp.int32, (8, 128), 0) == 0, counts_row, 0)
        cnt_rdmas = []
        for d in range(1, NZ):
            peer = (my_z + d) % NZ
            if COMM:
                cnt_rdma = pltpu.make_async_remote_copy(
                    src_ref=cnt_send_ref,
                    dst_ref=cnt_recv_ref.at[d - 1],
                    send_sem=cnt_send_sems.at[d - 1],
                    recv_sem=cnt_recv_sems.at[d - 1],
                    device_id=(my_x, my_y, peer),
                    device_id_type=pl.DeviceIdType.MESH,
                )
                cnt_rdma.start()
                cnt_rdmas.append(cnt_rdma)
            else:
                cnt_recv_ref[d - 1] = cnt_send_ref[...]

        NB, B = 8, n // 8
        tri_b = (lax.broadcasted_iota(jnp.int32, (B, B), 1)
                 < lax.broadcasted_iota(jnp.int32, (B, B), 0)
                 ).astype(jnp.bfloat16)
        base = jnp.zeros((1, 128), jnp.float32)
        rank_blocks = []
        for blk in range(NB):
            mask_blk = mask[blk * B:(blk + 1) * B, :]
            cum_blk = jnp.dot(tri_b, mask_blk.astype(jnp.bfloat16),
                              preferred_element_type=jnp.float32) + base
            rank_blocks.append(
                jnp.sum(jnp.where(mask_blk, cum_blk, 0.0),
                        axis=1, keepdims=True))
            base = base + jnp.sum(mask_blk.astype(jnp.float32),
                                  axis=0, keepdims=True)
        rank_c = jnp.concatenate(rank_blocks, axis=0).astype(jnp.int32)
        key_c = dest_c * P + rank_c
        iota_blk = lax.broadcasted_iota(jnp.int32, (n, P), 1)
        xbf_ref[...] = x_ref[...].astype(jnp.bfloat16)

        def pack_chunk(t):
            sel = (key_c == t * P + iota_blk).astype(jnp.bfloat16)
            chunk = lax.dot_general(
                sel, xbf_ref[...], dimension_numbers=(((0,), (0,)), ((), ())),
                preferred_element_type=jnp.float32)
            return chunk.astype(jnp.bfloat16)

        data_rdmas = {}
        for d in range(NZ - 1, 0, -1):
            peer = (my_z + d) % NZ
            send_ref[d - 1] = pack_chunk(peer)
            if COMM:
                data_rdma = pltpu.make_async_remote_copy(
                    src_ref=send_ref.at[d - 1],
                    dst_ref=recv_ref.at[d],
                    send_sem=data_send_sems.at[d - 1],
                    recv_sem=data_recv_sems.at[d - 1],
                    device_id=(my_x, my_y, peer),
                    device_id_type=pl.DeviceIdType.MESH,
                )
                data_rdma.start()
                data_rdmas[d] = data_rdma
            else:
                recv_ref[d] = send_ref[d - 1]

        recv_ref[0] = pack_chunk(my_z)

        for cnt_rdma in cnt_rdmas:
            cnt_rdma.wait()
        col_mask = lax.broadcasted_iota(jnp.int32, (8, 128), 1) == my_z
        row_mask = lax.broadcasted_iota(jnp.int32, (8, 128), 0) == 0
        lmask = col_mask & row_mask

        def _len_of(plane):
            return jnp.sum(jnp.where(lmask, plane, 0))

        l_by_d = [_len_of(cnt_send_ref[...])] + [
            _len_of(cnt_recv_ref[d - 1]) for d in range(1, NZ)
        ]

        len_src, slot_src = [], []
        for s in range(NZ):
            d_s = (my_z - s) % NZ
            ln = l_by_d[0]
            for d in range(1, NZ):
                ln = jnp.where(d_s == d, l_by_d[d], ln)
            len_src.append(ln)
            slot_src.append(d_s)
        starts = [jnp.int32(0)]
        for s in range(1, NZ):
            starts.append(starts[s - 1] + len_src[s - 1])

        j2 = lax.broadcasted_iota(jnp.int32, (n, 1), 0)
        s_idx = jnp.zeros((n, 1), jnp.int32)
        for s in range(1, NZ):
            s_idx = s_idx + (j2 >= starts[s]).astype(jnp.int32)
        start_j = jnp.full((n, 1), starts[0], jnp.int32)
        slot_j = jnp.full((n, 1), slot_src[0], jnp.int32)
        for s in range(1, NZ):
            sel_s = s_idx == s
            start_j = jnp.where(sel_s, starts[s], start_j)
            slot_j = jnp.where(sel_s, slot_src[s], slot_j)
        col_j = slot_j * P + (j2 - start_j)

        def partial(m):
            gsel = (col_j == m * P + iota_blk).astype(jnp.bfloat16)
            return jnp.dot(gsel, recv_ref[m],
                           preferred_element_type=jnp.float32
                           ).astype(jnp.bfloat16)

        out_ref[...] = partial(0)
        for d in range(1, NZ):
            if COMM:
                data_rdmas[d].wait()
            out_ref[...] = out_ref[...] + partial(d)

    return pl.pallas_call(
        body,
        out_shape=jax.ShapeDtypeStruct((n, ncol), jnp.bfloat16),
        in_specs=[
            pl.BlockSpec(memory_space=pltpu.VMEM),
            pl.BlockSpec(memory_space=pltpu.VMEM),
        ],
        out_specs=pl.BlockSpec(memory_space=pltpu.VMEM),
        scratch_shapes=[
            pltpu.VMEM((NZ - 1, P, N_COL), jnp.bfloat16),
            pltpu.VMEM((NZ, P, N_COL), jnp.bfloat16),
            pltpu.VMEM((N_ROWS, N_COL), jnp.bfloat16),
            pltpu.VMEM((8, 128), jnp.int32),
            pltpu.VMEM((NZ - 1, 8, 128), jnp.int32),
            pltpu.SemaphoreType.DMA((NZ - 1,)),
            pltpu.SemaphoreType.DMA((NZ - 1,)),
            pltpu.SemaphoreType.DMA((NZ - 1,)),
            pltpu.SemaphoreType.DMA((NZ - 1,)),
        ],
        compiler_params=pltpu.CompilerParams(
            collective_id=0 if COMM else None,
            vmem_limit_bytes=56 * 1024 * 1024),
    )(x, dest_col)


def kernel(x, dest):
    n, _ = x.shape
    return _a2av(x, dest.astype(jnp.int32).reshape(n, 1))
